# baseline (speedup 1.0000x reference)
import jax
import jax.numpy as jnp
from jax import lax
from jax.experimental import pallas as pl
from jax.experimental.pallas import tpu as pltpu
from jax.experimental.pallas import tpu_sc as plsc

_N = 50000
_B = 16384
_D = 256
_S = 10
_E = 256
_R = _S + 1
_DP = _D // 2

_INFO = plsc.get_sparse_core_info()
_NC = _INFO.num_cores
_NS = _INFO.num_subcores
_NW = _NC * _NS
_BPW = _B // _NW

_CB = 16
_G = _BPW // _CB
_NBUF = 4
_T = _G // _NBUF


_BM_PRE = 2000
_PRE_STEPS = _N // _BM_PRE


def _prepass_body(t_ref, w_ref, o_ref):
    t = t_ref[...].astype(jnp.bfloat16)
    w = w_ref[...].astype(jnp.bfloat16)
    lo = jnp.dot(t, w[:, :_DP], preferred_element_type=jnp.float32)
    hi = jnp.dot(t, w[:, _DP:], preferred_element_type=jnp.float32)
    lo_t = lax.shift_right_logical(
        lax.bitcast_convert_type(lo, jnp.int32), 16)
    hi_t = lax.bitcast_convert_type(hi, jnp.int32) & jnp.int32(-65536)
    o_ref[...] = hi_t | lo_t


def _prepass(table, W):
    return pl.pallas_call(
        _prepass_body,
        grid=(2 * _PRE_STEPS,),
        in_specs=[
            pl.BlockSpec((_BM_PRE, _D), lambda i: (i % _PRE_STEPS, 0)),
            pl.BlockSpec((_D, _E), lambda i: (1 - i // _PRE_STEPS, 0)),
        ],
        out_specs=pl.BlockSpec((_BM_PRE, _DP), lambda i: (i, 0)),
        out_shape=jax.ShapeDtypeStruct((2 * _N, _DP), jnp.int32),
    )(table, W)


def _sc_body(tpk, nodes_hbm, nbr_hbm, bias_hbm, out_h, nv, bv, bias_v,
             nbuf, bbuf, hbuf, sem_g0, sem_g1, sem_g2, sem_g3, sem_o):
    sems_g = (sem_g0, sem_g1, sem_g2, sem_g3)
    wid = lax.axis_index("s") * _NC + lax.axis_index("c")
    base = pl.multiple_of(wid * _BPW, _BPW)

    pltpu.sync_copy(nodes_hbm.at[pl.ds(base, _BPW)], nv)
    pltpu.sync_copy(nbr_hbm.at[pl.ds(base * _S, _BPW * _S)], bv)
    pltpu.sync_copy(bias_hbm, bias_v)
    off_n = jnp.int32(_N)
    for r in range(_BPW // 16):
        sl = pl.ds(r * 16, 16)
        nv[sl] = nv[sl] + off_n
    blo_regs = [bias_v[pl.ds(16 * d, 16)] for d in range(_DP // 16)]
    bhi_regs = [bias_v[pl.ds(_DP + 16 * d, 16)] for d in range(_DP // 16)]

    _HB = _CB * _S // 2

    def gather_srcs(g):
        offb = pl.multiple_of(g * (_CB * _S), _CB * _S)
        offn = pl.multiple_of(g * _CB, _CB)
        return (tpk.at[bv.at[pl.ds(offb, _HB)]],
                tpk.at[bv.at[pl.ds(offb + _HB, _HB)]],
                tpk.at[nv.at[pl.ds(offn, _CB)]])

    def fire(g, b):
        src0, src1, srcn = gather_srcs(g)
        pltpu.async_copy(src0, bbuf.at[b, pl.ds(0, _HB)], sems_g[b])
        pltpu.async_copy(src1, bbuf.at[b, pl.ds(_HB, _HB)], sems_g[b])
        pltpu.async_copy(srcn, nbuf.at[b], sems_g[b])

    hi_mask = jnp.int32(-65536)
    inv_s = jnp.float32(1.0 / _S)

    def process(g, b, fire_next):
        src0, src1, srcn = gather_srcs(g)
        pltpu.make_async_copy(src0, bbuf.at[b, pl.ds(0, _HB)],
                              sems_g[b]).wait()
        pltpu.make_async_copy(src1, bbuf.at[b, pl.ds(_HB, _HB)],
                              sems_g[b]).wait()
        pltpu.make_async_copy(srcn, nbuf.at[b], sems_g[b]).wait()

        def row(i, _):
            r0 = i * _S
            for d in range(_DP // 16):
                sl = pl.ds(d * 16, 16)
                v = bbuf[b, r0, sl]
                alo = lax.bitcast_convert_type(v << 16, jnp.float32)
                ahi = lax.bitcast_convert_type(v & hi_mask, jnp.float32)
                for s in range(1, _S):
                    v = bbuf[b, r0 + s, sl]
                    alo = alo + lax.bitcast_convert_type(v << 16, jnp.float32)
                    ahi = ahi + lax.bitcast_convert_type(v & hi_mask,
                                                         jnp.float32)
                vn = nbuf[b, i, sl]
                alo = (alo * inv_s + blo_regs[d]
                       + lax.bitcast_convert_type(vn << 16, jnp.float32))
                ahi = (ahi * inv_s + bhi_regs[d]
                       + lax.bitcast_convert_type(vn & hi_mask, jnp.float32))
                hbuf[i, pl.ds(d * 16, 16)] = jnp.maximum(alo, 0.0)
                hbuf[i, pl.ds(_DP + d * 16, 16)] = jnp.maximum(ahi, 0.0)
            return _
        lax.fori_loop(0, _CB, row, None)

        if fire_next is not None:
            fire(fire_next, b)

        cp = pltpu.async_copy(hbuf, out_h.at[pl.ds(base + g * _CB, _CB)],
                              sem_o)
        cp.wait()

    for b in range(_NBUF):
        fire(b, b)

    def outer(t, _):
        g0 = t * _NBUF
        for b in range(_NBUF):
            process(g0 + b, b, g0 + b + _NBUF)
        return _
    lax.fori_loop(0, _T - 1, outer, None)

    for b in range(_NBUF):
        process((_T - 1) * _NBUF + b, b, None)


_sc_encode = pl.kernel(
    _sc_body,
    out_type=jax.ShapeDtypeStruct((_B, _E), jnp.float32),
    mesh=plsc.VectorSubcoreMesh(core_axis_name="c", subcore_axis_name="s"),
    scratch_types=[
        pltpu.VMEM((_BPW,), jnp.int32),
        pltpu.VMEM((_BPW * _S,), jnp.int32),
        pltpu.VMEM((_E,), jnp.float32),
        pltpu.VMEM((_NBUF, _CB, _DP), jnp.int32),
        pltpu.VMEM((_NBUF, _CB * _S, _DP), jnp.int32),
        pltpu.VMEM((_CB, _E), jnp.float32),
        pltpu.SemaphoreType.DMA,
        pltpu.SemaphoreType.DMA,
        pltpu.SemaphoreType.DMA,
        pltpu.SemaphoreType.DMA,
        pltpu.SemaphoreType.DMA,
    ],
)


def kernel(feature_table, nodes, neighbor_idx, W, b):
    tpk = _prepass(feature_table, W)
    return _sc_encode(tpk, nodes.astype(jnp.int32),
                      neighbor_idx.astype(jnp.int32).reshape(-1), b)


# speedup vs baseline: 1.0236x; 1.0236x over previous
import jax
import jax.numpy as jnp
from jax import lax
from jax.experimental import pallas as pl
from jax.experimental.pallas import tpu as pltpu
from jax.experimental.pallas import tpu_sc as plsc

_N = 50000
_B = 16384
_D = 256
_S = 10
_E = 256
_R = _S + 1
_DP = _D // 2

_INFO = plsc.get_sparse_core_info()
_NC = _INFO.num_cores
_NS = _INFO.num_subcores
_NW = _NC * _NS
_BPW = _B // _NW

_CB = 16
_G = _BPW // _CB
_NBUF = 4
_T = _G // _NBUF


_BM_PRE = 2000
_PRE_STEPS = _N // _BM_PRE


def _prepass_body(t_ref, w_ref, b_ref, o_ref):
    t = t_ref[...].astype(jnp.bfloat16)
    w = w_ref[...].astype(jnp.bfloat16)
    bb = b_ref[0]
    lo = jnp.dot(t, w[:, :_DP], preferred_element_type=jnp.float32) + bb[:, :_DP]
    hi = jnp.dot(t, w[:, _DP:], preferred_element_type=jnp.float32) + bb[:, _DP:]
    lo_t = lax.shift_right_logical(
        lax.bitcast_convert_type(lo, jnp.int32), 16)
    hi_t = lax.bitcast_convert_type(hi, jnp.int32) & jnp.int32(-65536)
    o_ref[...] = hi_t | lo_t


def _prepass(table, W, b2):
    return pl.pallas_call(
        _prepass_body,
        grid=(2 * _PRE_STEPS,),
        in_specs=[
            pl.BlockSpec((_BM_PRE, _D), lambda i: (i % _PRE_STEPS, 0)),
            pl.BlockSpec((_D, _E), lambda i: (1 - i // _PRE_STEPS, 0)),
            pl.BlockSpec((1, 1, _E), lambda i: (i // _PRE_STEPS, 0, 0)),
        ],
        out_specs=pl.BlockSpec((_BM_PRE, _DP), lambda i: (i, 0)),
        out_shape=jax.ShapeDtypeStruct((2 * _N, _DP), jnp.int32),
    )(table, W, b2)


def _sc_body(tpk, nodes_hbm, nbr_hbm, out_h, nv, bv,
             nbuf, bbuf, hbuf, sem_g0, sem_g1, sem_g2, sem_g3, sem_o):
    sems_g = (sem_g0, sem_g1, sem_g2, sem_g3)
    wid = lax.axis_index("s") * _NC + lax.axis_index("c")
    base = pl.multiple_of(wid * _BPW, _BPW)

    pltpu.sync_copy(nodes_hbm.at[pl.ds(base, _BPW)], nv)
    pltpu.sync_copy(nbr_hbm.at[pl.ds(base * _S, _BPW * _S)], bv)
    off_n = jnp.int32(_N)
    for r in range(_BPW // 16):
        sl = pl.ds(r * 16, 16)
        nv[sl] = nv[sl] + off_n

    _HB = _CB * _S // 2

    def gather_srcs(g):
        offb = pl.multiple_of(g * (_CB * _S), _CB * _S)
        offn = pl.multiple_of(g * _CB, _CB)
        return (tpk.at[bv.at[pl.ds(offb, _HB)]],
                tpk.at[bv.at[pl.ds(offb + _HB, _HB)]],
                tpk.at[nv.at[pl.ds(offn, _CB)]])

    def fire(g, b):
        src0, src1, srcn = gather_srcs(g)
        pltpu.async_copy(src0, bbuf.at[b, pl.ds(0, _HB)], sems_g[b])
        pltpu.async_copy(src1, bbuf.at[b, pl.ds(_HB, _HB)], sems_g[b])
        pltpu.async_copy(srcn, nbuf.at[b], sems_g[b])

    hi_mask = jnp.int32(-65536)
    inv_s = jnp.float32(1.0 / _S)

    def process(g, b, fire_next):
        src0, src1, srcn = gather_srcs(g)
        pltpu.make_async_copy(src0, bbuf.at[b, pl.ds(0, _HB)],
                              sems_g[b]).wait()
        pltpu.make_async_copy(src1, bbuf.at[b, pl.ds(_HB, _HB)],
                              sems_g[b]).wait()
        pltpu.make_async_copy(srcn, nbuf.at[b], sems_g[b]).wait()

        def row(i, _):
            r0 = i * _S
            for d in range(_DP // 16):
                sl = pl.ds(d * 16, 16)
                v = bbuf[b, r0, sl]
                alo = lax.bitcast_convert_type(v << 16, jnp.float32)
                ahi = lax.bitcast_convert_type(v & hi_mask, jnp.float32)
                for s in range(1, _S):
                    v = bbuf[b, r0 + s, sl]
                    alo = alo + lax.bitcast_convert_type(v << 16, jnp.float32)
                    ahi = ahi + lax.bitcast_convert_type(v & hi_mask,
                                                         jnp.float32)
                vn = nbuf[b, i, sl]
                alo = alo * inv_s + lax.bitcast_convert_type(vn << 16,
                                                             jnp.float32)
                ahi = ahi * inv_s + lax.bitcast_convert_type(vn & hi_mask,
                                                             jnp.float32)
                hbuf[i, pl.ds(d * 16, 16)] = jnp.maximum(alo, 0.0)
                hbuf[i, pl.ds(_DP + d * 16, 16)] = jnp.maximum(ahi, 0.0)
            return _
        lax.fori_loop(0, _CB, row, None)

        if fire_next is not None:
            fire(fire_next, b)

        cp = pltpu.async_copy(hbuf, out_h.at[pl.ds(base + g * _CB, _CB)],
                              sem_o)
        cp.wait()

    for b in range(_NBUF):
        fire(b, b)

    def outer(t, _):
        g0 = t * _NBUF
        for b in range(_NBUF):
            process(g0 + b, b, g0 + b + _NBUF)
        return _
    lax.fori_loop(0, _T - 1, outer, None)

    for b in range(_NBUF):
        process((_T - 1) * _NBUF + b, b, None)


_sc_encode = pl.kernel(
    _sc_body,
    out_type=jax.ShapeDtypeStruct((_B, _E), jnp.float32),
    mesh=plsc.VectorSubcoreMesh(core_axis_name="c", subcore_axis_name="s"),
    scratch_types=[
        pltpu.VMEM((_BPW,), jnp.int32),
        pltpu.VMEM((_BPW * _S,), jnp.int32),
        pltpu.VMEM((_NBUF, _CB, _DP), jnp.int32),
        pltpu.VMEM((_NBUF, _CB * _S, _DP), jnp.int32),
        pltpu.VMEM((_CB, _E), jnp.float32),
        pltpu.SemaphoreType.DMA,
        pltpu.SemaphoreType.DMA,
        pltpu.SemaphoreType.DMA,
        pltpu.SemaphoreType.DMA,
        pltpu.SemaphoreType.DMA,
    ],
)


def kernel(feature_table, nodes, neighbor_idx, W, b):
    b1 = b.reshape(1, _E)
    b2 = jnp.stack([jnp.zeros_like(b1), b1])
    tpk = _prepass(feature_table, W, b2)
    return _sc_encode(tpk, nodes.astype(jnp.int32),
                      neighbor_idx.astype(jnp.int32).reshape(-1))


# speedup vs baseline: 1.0407x; 1.0167x over previous
import jax
import jax.numpy as jnp
from jax import lax
from jax.experimental import pallas as pl
from jax.experimental.pallas import tpu as pltpu
from jax.experimental.pallas import tpu_sc as plsc

_N = 50000
_B = 16384
_D = 256
_S = 10
_E = 256
_R = _S + 1
_DP = _D // 2

_INFO = plsc.get_sparse_core_info()
_NC = _INFO.num_cores
_NS = _INFO.num_subcores
_NW = _NC * _NS
_BPW = _B // _NW

_CB = 32
_G = _BPW // _CB
_NBUF = 2
_T = _G // _NBUF


_BM_PRE = 2000
_PRE_STEPS = _N // _BM_PRE


def _prepass_body(t_ref, w_ref, b_ref, o_ref):
    pid = pl.program_id(0)
    t = t_ref[...].astype(jnp.bfloat16)
    w = w_ref[...].astype(jnp.bfloat16)
    scale = jnp.where(pid >= _PRE_STEPS, 1.0, 0.0).astype(jnp.float32)
    lo = (jnp.dot(t, w[:, :_DP], preferred_element_type=jnp.float32)
          + b_ref[:, :_DP] * scale).astype(jnp.bfloat16)
    hi = (jnp.dot(t, w[:, _DP:], preferred_element_type=jnp.float32)
          + b_ref[:, _DP:] * scale).astype(jnp.bfloat16)
    lo_u = lax.bitcast_convert_type(lo, jnp.uint16).astype(jnp.int32)
    hi_u = lax.bitcast_convert_type(hi, jnp.uint16).astype(jnp.int32)
    o_ref[...] = (hi_u << 16) | lo_u


def _prepass(table, W, b2):
    return pl.pallas_call(
        _prepass_body,
        grid=(2 * _PRE_STEPS,),
        in_specs=[
            pl.BlockSpec((_BM_PRE, _D), lambda i: (i % _PRE_STEPS, 0)),
            pl.BlockSpec((_D, _E), lambda i: (1 - i // _PRE_STEPS, 0)),
            pl.BlockSpec((1, _E), lambda i: (0, 0)),
        ],
        out_specs=pl.BlockSpec((_BM_PRE, _DP), lambda i: (i, 0)),
        out_shape=jax.ShapeDtypeStruct((2 * _N, _DP), jnp.int32),
    )(table, W, b2)


def _sc_body(tpk, nodes_hbm, nbr_hbm, out_h, nv, bv, nbuf, bbuf, hbuf,
             sem_g0, sem_g1, sem_g2, sem_g3, sem_o):
    sems_g = (sem_g0, sem_g1, sem_g2, sem_g3)
    wid = lax.axis_index("s") * _NC + lax.axis_index("c")
    base = pl.multiple_of(wid * _BPW, _BPW)

    pltpu.sync_copy(nodes_hbm.at[pl.ds(base, _BPW)], nv)
    pltpu.sync_copy(nbr_hbm.at[pl.ds(base * _S, _BPW * _S)], bv)
    off_n = jnp.int32(_N)
    for r in range(_BPW // 16):
        sl = pl.ds(r * 16, 16)
        nv[sl] = nv[sl] + off_n

    _HB = 80
    _NSUB = _CB * _S // _HB

    def gather_srcs(g):
        offb = pl.multiple_of(g * (_CB * _S), _CB * _S)
        offn = pl.multiple_of(g * _CB, _CB)
        subs = [tpk.at[bv.at[pl.ds(offb + k * _HB, _HB)]]
                for k in range(_NSUB)]
        return subs, tpk.at[nv.at[pl.ds(offn, _CB)]]

    def fire(g, b):
        subs, srcn = gather_srcs(g)
        for k in range(_NSUB):
            pltpu.async_copy(subs[k], bbuf.at[b, pl.ds(k * _HB, _HB)],
                             sems_g[b])
        pltpu.async_copy(srcn, nbuf.at[b], sems_g[b])

    hi_mask = jnp.int32(-65536)
    inv_s = jnp.float32(1.0 / _S)

    def process(g, b, fire_next):
        subs, srcn = gather_srcs(g)
        for k in range(_NSUB):
            pltpu.make_async_copy(subs[k], bbuf.at[b, pl.ds(k * _HB, _HB)],
                                  sems_g[b]).wait()
        pltpu.make_async_copy(srcn, nbuf.at[b], sems_g[b]).wait()

        def row(i, _):
            r0 = i * _S
            for d in range(_DP // 16):
                sl = pl.ds(d * 16, 16)
                v = bbuf[b, r0, sl]
                alo = lax.bitcast_convert_type(v << 16, jnp.float32)
                ahi = lax.bitcast_convert_type(v & hi_mask, jnp.float32)
                for s in range(1, _S):
                    v = bbuf[b, r0 + s, sl]
                    alo = alo + lax.bitcast_convert_type(v << 16, jnp.float32)
                    ahi = ahi + lax.bitcast_convert_type(v & hi_mask,
                                                         jnp.float32)
                vn = nbuf[b, i, sl]
                alo = alo * inv_s + lax.bitcast_convert_type(vn << 16,
                                                             jnp.float32)
                ahi = ahi * inv_s + lax.bitcast_convert_type(vn & hi_mask,
                                                             jnp.float32)
                hbuf[i, pl.ds(d * 16, 16)] = jnp.maximum(alo, 0.0)
                hbuf[i, pl.ds(_DP + d * 16, 16)] = jnp.maximum(ahi, 0.0)
            return _
        lax.fori_loop(0, _CB, row, None)

        if fire_next is not None:
            fire(fire_next, b)

        cp = pltpu.async_copy(hbuf, out_h.at[pl.ds(base + g * _CB, _CB)],
                              sem_o)
        cp.wait()

    for b in range(_NBUF):
        fire(b, b)

    def outer(t, _):
        g0 = t * _NBUF
        for b in range(_NBUF):
            process(g0 + b, b, g0 + b + _NBUF)
        return _
    lax.fori_loop(0, _T - 1, outer, None)

    for b in range(_NBUF):
        process((_T - 1) * _NBUF + b, b, None)


_sc_encode = pl.kernel(
    _sc_body,
    out_type=jax.ShapeDtypeStruct((_B, _E), jnp.float32),
    mesh=plsc.VectorSubcoreMesh(core_axis_name="c", subcore_axis_name="s"),
    scratch_types=[
        pltpu.VMEM((_BPW,), jnp.int32),
        pltpu.VMEM((_BPW * _S,), jnp.int32),
        pltpu.VMEM((_NBUF, _CB, _DP), jnp.int32),
        pltpu.VMEM((_NBUF, _CB * _S, _DP), jnp.int32),
        pltpu.VMEM((_CB, _E), jnp.float32),
        pltpu.SemaphoreType.DMA,
        pltpu.SemaphoreType.DMA,
        pltpu.SemaphoreType.DMA,
        pltpu.SemaphoreType.DMA,
        pltpu.SemaphoreType.DMA,
    ],
)


def kernel(feature_table, nodes, neighbor_idx, W, b):
    tpk = _prepass(feature_table, W, b.reshape(1, _E))
    return _sc_encode(tpk, nodes.astype(jnp.int32),
                      neighbor_idx.astype(jnp.int32).reshape(-1))
